# baseline (speedup 1.0000x reference)
import functools

import jax
import jax.numpy as jnp
from jax import lax
from jax.experimental import pallas as pl
from jax.experimental.pallas import tpu as pltpu
from jax.experimental.pallas import tpu_sc as plsc

B = 16384
D = 16
NW = 32
BPW = B // NW
CHUNK = 128
NCHUNK = BPW // CHUNK


def _sigmoid(x):
    return 1.0 / (1.0 + jnp.exp(-x))


@functools.partial(
    pl.kernel,
    out_type=jax.ShapeDtypeStruct((B, 2), jnp.float32),
    mesh=plsc.VectorSubcoreMesh(core_axis_name="c", subcore_axis_name="s"),
    compiler_params=pltpu.CompilerParams(
        needs_layout_passes=False, use_tc_tiling_on_sc=False),
    scratch_types=[
        pltpu.VMEM((BPW,), jnp.int32),
        pltpu.VMEM((BPW,), jnp.int32),
        pltpu.VMEM((BPW,), jnp.int32),
        pltpu.VMEM((BPW, D), jnp.float32),
        pltpu.VMEM((BPW, D), jnp.float32),
        pltpu.VMEM((BPW, D), jnp.float32),
        pltpu.VMEM((D,), jnp.float32),
        pltpu.VMEM((16,), jnp.float32),
        pltpu.VMEM((BPW, 2), jnp.float32),
        pltpu.SemaphoreType.DMA,
    ],
)
def _lightgcn_sc(user_hbm, pos_hbm, neg_hbm, ut_hbm, it_hbm, w_hbm, b_hbm,
                 out_hbm, idx_u, idx_p, idx_n, rows_u, rows_p, rows_n,
                 w_v, b_v, out_v, sem):
    wid = lax.axis_index("s") * 2 + lax.axis_index("c")
    base = wid * BPW

    pltpu.sync_copy(user_hbm.at[pl.ds(base, BPW)], idx_u)
    pltpu.sync_copy(pos_hbm.at[pl.ds(base, BPW)], idx_p)
    pltpu.sync_copy(neg_hbm.at[pl.ds(base, BPW)], idx_n)
    pltpu.sync_copy(w_hbm, w_v)
    pltpu.sync_copy(b_hbm, b_v)

    copies = []
    for j in range(NCHUNK):
        sl = pl.ds(j * CHUNK, CHUNK)
        copies.append(pltpu.async_copy(ut_hbm.at[idx_u.at[sl]], rows_u.at[sl], sem))
        copies.append(pltpu.async_copy(it_hbm.at[idx_p.at[sl]], rows_p.at[sl], sem))
        copies.append(pltpu.async_copy(it_hbm.at[idx_n.at[sl]], rows_n.at[sl], sem))
    for cp in copies:
        cp.wait()

    lane = lax.iota(jnp.int32, 16)
    col0 = jnp.zeros((16,), jnp.int32)
    col1 = jnp.ones((16,), jnp.int32)
    bias_vec = b_v[...]
    wvec = w_v[...]

    def block_body(blk, _):
        rows = blk * 16 + lane
        pos_acc = bias_vec
        neg_acc = bias_vec
        for d in range(D):
            cold = jnp.full((16,), d, jnp.int32)
            u = plsc.load_gather(rows_u, [rows, cold])
            p = plsc.load_gather(rows_p, [rows, cold])
            n = plsc.load_gather(rows_n, [rows, cold])
            wd = wvec[d]
            pos_acc = pos_acc + _sigmoid(u * p) * wd
            neg_acc = neg_acc + _sigmoid(u * n) * wd
        plsc.store_scatter(out_v, [rows, col0], pos_acc)
        plsc.store_scatter(out_v, [rows, col1], neg_acc)
        return _

    lax.fori_loop(0, BPW // 16, block_body, None)

    pltpu.sync_copy(out_v, out_hbm.at[pl.ds(base, BPW)])


def kernel(user, pos, neg, user_table, item_table, W, b):
    user = jnp.asarray(user, jnp.int32).reshape(B)
    pos = jnp.asarray(pos, jnp.int32).reshape(B)
    neg = jnp.asarray(neg, jnp.int32).reshape(B)
    w = W.reshape(D)
    b16 = jnp.broadcast_to(b.reshape(1), (16,)).astype(jnp.float32)
    return _lightgcn_sc(user, pos, neg, user_table, item_table, w, b16)
